# baseline (speedup 1.0000x reference)
import functools

import jax
import jax.numpy as jnp
from jax import lax
from jax.experimental import pallas as pl
from jax.experimental.pallas import tpu as pltpu
from jax.experimental.pallas import tpu_sc as plsc

INPUT_SIZE = 1024
HIDDEN_SIZE = 176
OUTPUT_SIZE = 1024
NUM_EXPERTS = 64
NUM_SELECTS = 2
BALANCE_LOSS_WEIGHT = 1e-2

_GATE_BLK = 512
_BR = 256

_NC = 2
_NS = 16
_NW = _NC * _NS
_DCH = 64
_CCH = 32


def _gate_body(x_ref, g1_ref, g2_ref, idx_ref, sc_ref, rank_ref, cnt_ref,
               loss_ref, imp_ref, load_ref, carry_ref):
    tb = pl.program_id(0)
    nb = pl.num_programs(0)
    x = x_ref[...]
    t1 = jnp.tanh(jax.lax.dot_general(x, g1_ref[...], (((1,), (0,)), ((), ()))))
    logits = jax.lax.dot_general(t1, g2_ref[...], (((1,), (0,)), ((), ())))
    lane = jax.lax.broadcasted_iota(jnp.int32, logits.shape, 1)
    m1 = jnp.max(logits, axis=1, keepdims=True)
    i1 = jnp.min(jnp.where(logits == m1, lane, NUM_EXPERTS), axis=1, keepdims=True)
    mask1 = lane == i1
    l2 = jnp.where(mask1, -jnp.inf, logits)
    m2 = jnp.max(l2, axis=1, keepdims=True)
    i2 = jnp.min(jnp.where(l2 == m2, lane, NUM_EXPERTS), axis=1, keepdims=True)
    mask2 = lane == i2
    s1 = 1.0 / (1.0 + jnp.exp(m2 - m1))
    h1 = mask1.astype(jnp.float32)
    h2 = mask2.astype(jnp.float32)
    idx_ref[...] = jnp.concatenate([i1, i2], axis=1)
    sc_ref[...] = jnp.concatenate([s1, 1.0 - s1], axis=1)

    @pl.when(tb == 0)
    def _init():
        imp_ref[...] = jnp.zeros_like(imp_ref)
        load_ref[...] = jnp.zeros_like(load_ref)
        carry_ref[...] = jnp.zeros_like(carry_ref)

    r_io = jax.lax.broadcasted_iota(jnp.int32, (_GATE_BLK, _GATE_BLK), 0)
    c_io = jax.lax.broadcasted_iota(jnp.int32, (_GATE_BLK, _GATE_BLK), 1)
    ltri = (c_io < r_io).astype(jnp.bfloat16)
    hsum = h1 + h2
    cum = jax.lax.dot_general(ltri, hsum.astype(jnp.bfloat16),
                              (((1,), (0,)), ((), ())),
                              preferred_element_type=jnp.float32)
    carry = carry_ref[...]
    rank1 = jnp.sum(jnp.where(mask1, carry + cum, 0.0), axis=1, keepdims=True)
    rank2 = jnp.sum(jnp.where(mask2, carry + cum + h1, 0.0), axis=1,
                    keepdims=True)
    rank_ref[...] = jnp.concatenate([rank1, rank2], axis=1).astype(jnp.int32)
    carry_ref[...] = carry + jnp.sum(hsum, axis=0, keepdims=True)

    sf1 = jnp.where(mask1, s1, 0.0)
    sf2 = jnp.where(mask2, 1.0 - s1, 0.0)
    imp_ref[...] += jnp.sum(sf1 + sf2, axis=0, keepdims=True)
    load_ref[...] += (jnp.sum((sf1 > 0.0).astype(jnp.float32), axis=0, keepdims=True)
                      + jnp.sum((sf2 > 0.0).astype(jnp.float32), axis=0, keepdims=True))

    @pl.when(tb == nb - 1)
    def _fin():
        def cv2(v):
            mean = jnp.mean(v)
            var = jnp.sum((v - mean) ** 2) / (NUM_EXPERTS - 1)
            return var / (mean * mean + 1e-10)

        loss = BALANCE_LOSS_WEIGHT * (cv2(imp_ref[...]) + cv2(load_ref[...]))
        loss_ref[...] = loss * jnp.ones((1, 1), jnp.float32)
        cnt_ref[...] = carry_ref[...].astype(jnp.int32)


def _group_body(tb, te, tf, rl, rh, xs_ref, wg_ref, wu_ref, wd_ref,
                bg_ref, bu_ref, bd_ref, out_ref):
    i = pl.program_id(0)
    x = xs_ref[...]
    g = jax.lax.dot_general(x, wg_ref[0], (((1,), (1,)), ((), ()))) + bg_ref[0]
    u = jax.lax.dot_general(x, wu_ref[0], (((1,), (1,)), ((), ()))) + bu_ref[0]
    h = (g / (1.0 + jnp.exp(-g))) * u
    out = jax.lax.dot_general(h, wd_ref[0], (((1,), (1,)), ((), ()))) + bd_ref[0]
    rows = tb[i] * _BR + jax.lax.broadcasted_iota(jnp.int32, (_BR, 1), 0)
    m = ((rows >= rl[i]) & (rows < rh[i])).astype(jnp.float32)
    contrib = m * out

    @pl.when(tf[i] == 1)
    def _set():
        out_ref[...] = contrib

    @pl.when(tf[i] == 0)
    def _acc():
        out_ref[...] += contrib


def _routing_metadata(idx, sc, rank, cnt):
    A = idx.size
    counts = cnt.reshape(NUM_EXPERTS)
    bounds = jnp.concatenate(
        [jnp.zeros(1, jnp.int32), jnp.cumsum(counts, dtype=jnp.int32)])
    starts, ends = bounds[:NUM_EXPERTS], bounds[1:]
    sel = idx[..., None] == jnp.arange(NUM_EXPERTS, dtype=jnp.int32)
    pos = jnp.sum(jnp.where(sel, starts[None, None, :], 0),
                  axis=2).astype(jnp.int32) + rank
    NB = A // _BR
    edges0 = jnp.arange(NB, dtype=jnp.int32) * _BR
    edges1 = edges0 + (_BR - 1)
    eb_first = jnp.sum((ends[None, :] <= edges0[:, None]).astype(jnp.int32), axis=1)
    eb_last = jnp.sum((ends[None, :] <= edges1[:, None]).astype(jnp.int32), axis=1)
    nb = eb_last - eb_first + 1
    cum = jnp.concatenate([jnp.zeros(1, jnp.int32),
                           jnp.cumsum(nb, dtype=jnp.int32)])
    total = cum[NB]
    NT = NB + NUM_EXPERTS
    ti = jnp.arange(NT, dtype=jnp.int32)
    blk = jnp.clip(jnp.searchsorted(cum, ti, side="right") - 1, 0, NB - 1).astype(jnp.int32)
    pad = ti >= total
    tile_block = jnp.where(pad, NB - 1, blk)
    tile_expert = jnp.where(pad, eb_last[NB - 1],
                            eb_first[blk] + (ti - cum[blk]))
    row_lo = jnp.where(pad, 0, jnp.maximum(jnp.take(starts, tile_expert),
                                           tile_block * _BR))
    row_hi = jnp.where(pad, 0, jnp.minimum(jnp.take(ends, tile_expert),
                                           (tile_block + 1) * _BR))
    tile_first = ((ti == cum[blk]) & ~pad).astype(jnp.int32)
    return pos, tile_block, tile_expert, tile_first, row_lo, row_hi


def _dispatch_body(xf_hbm, pA_hbm, pB_hbm, xs_hbm, idxA_v, idxB_v, rows_v,
                   semA, semB):
    wid = lax.axis_index("s") * _NC + lax.axis_index("c")
    base = wid * (_DCH * (pA_hbm.shape[1]))
    pltpu.sync_copy(pA_hbm.at[wid], idxA_v)
    pltpu.sync_copy(pB_hbm.at[wid], idxB_v)
    for c in range(pA_hbm.shape[1]):
        pltpu.sync_copy(xf_hbm.at[pl.ds(base + c * _DCH, _DCH)], rows_v)
        a = pltpu.make_async_copy(rows_v, xs_hbm.at[idxA_v.at[c]], semA)
        b = pltpu.make_async_copy(rows_v, xs_hbm.at[idxB_v.at[c]], semB)
        a.start()
        b.start()
        a.wait()
        b.wait()


def _combine_body(outs_hbm, pA_hbm, pB_hbm, sA_hbm, sB_hbm, y_hbm,
                  idxA_v, idxB_v, sA_v, sB_v, a_v, b_v, semA, semB):
    wid = lax.axis_index("s") * _NC + lax.axis_index("c")
    nch = pA_hbm.shape[1]
    base = wid * (_CCH * nch)
    pltpu.sync_copy(pA_hbm.at[wid], idxA_v)
    pltpu.sync_copy(pB_hbm.at[wid], idxB_v)
    pltpu.sync_copy(sA_hbm.at[wid], sA_v)
    pltpu.sync_copy(sB_hbm.at[wid], sB_v)

    def chunk(c, carry):
        ga = pltpu.make_async_copy(outs_hbm.at[idxA_v.at[c]], a_v, semA)
        gb = pltpu.make_async_copy(outs_hbm.at[idxB_v.at[c]], b_v, semB)
        ga.start()
        gb.start()
        ga.wait()
        gb.wait()
        def col(k, cc):
            sl = pl.ds(k * 16, 16)
            for r in range(_CCH):
                a_v[r, sl] = (a_v[r, sl] * sA_v[c, r, :]
                              + b_v[r, sl] * sB_v[c, r, :])
            return cc

        lax.fori_loop(0, OUTPUT_SIZE // 16, col, 0)
        pltpu.sync_copy(a_v, y_hbm.at[pl.ds(base + c * _CCH, _CCH)])
        return carry

    lax.fori_loop(0, nch, chunk, 0)


def kernel(x, gate_w1, gate_w2, W_gate, W_up, W_down, b_gate, b_up, b_down):
    B, S, _ = x.shape
    T = B * S
    A = T * NUM_SELECTS
    xf = x.reshape(T, INPUT_SIZE)

    nbg = T // _GATE_BLK
    idx, sc, rank, cnt, loss = pl.pallas_call(
        _gate_body,
        grid=(nbg,),
        in_specs=[
            pl.BlockSpec((_GATE_BLK, INPUT_SIZE), lambda i: (i, 0)),
            pl.BlockSpec((INPUT_SIZE, NUM_EXPERTS), lambda i: (0, 0)),
            pl.BlockSpec((NUM_EXPERTS, NUM_EXPERTS), lambda i: (0, 0)),
        ],
        out_specs=[
            pl.BlockSpec((_GATE_BLK, NUM_SELECTS), lambda i: (i, 0)),
            pl.BlockSpec((_GATE_BLK, NUM_SELECTS), lambda i: (i, 0)),
            pl.BlockSpec((_GATE_BLK, NUM_SELECTS), lambda i: (i, 0)),
            pl.BlockSpec((1, NUM_EXPERTS), lambda i: (0, 0)),
            pl.BlockSpec((1, 1), lambda i: (0, 0)),
        ],
        out_shape=[
            jax.ShapeDtypeStruct((T, NUM_SELECTS), jnp.int32),
            jax.ShapeDtypeStruct((T, NUM_SELECTS), jnp.float32),
            jax.ShapeDtypeStruct((T, NUM_SELECTS), jnp.int32),
            jax.ShapeDtypeStruct((1, NUM_EXPERTS), jnp.int32),
            jax.ShapeDtypeStruct((1, 1), jnp.float32),
        ],
        scratch_shapes=[
            pltpu.VMEM((1, NUM_EXPERTS), jnp.float32),
            pltpu.VMEM((1, NUM_EXPERTS), jnp.float32),
            pltpu.VMEM((1, NUM_EXPERTS), jnp.float32),
        ],
    )(xf, gate_w1, gate_w2)

    (pos, tile_block, tile_expert, tile_first, row_lo,
     row_hi) = _routing_metadata(idx, sc, rank, cnt)

    mesh = plsc.VectorSubcoreMesh(core_axis_name="c", subcore_axis_name="s")
    ndch = T // (_NW * _DCH)
    pA3 = pos[:, 0].reshape(_NW, ndch, _DCH)
    pB3 = pos[:, 1].reshape(_NW, ndch, _DCH)
    xs = pl.kernel(
        _dispatch_body,
        out_type=jax.ShapeDtypeStruct((A, INPUT_SIZE), jnp.float32),
        mesh=mesh,
        compiler_params=pltpu.CompilerParams(use_tc_tiling_on_sc=False),
        scratch_types=[
            pltpu.VMEM((ndch, _DCH), jnp.int32),
            pltpu.VMEM((ndch, _DCH), jnp.int32),
            pltpu.VMEM((_DCH, INPUT_SIZE), jnp.float32),
            pltpu.SemaphoreType.DMA,
            pltpu.SemaphoreType.DMA,
        ],
    )(xf, pA3, pB3)

    NT = A // _BR + NUM_EXPERTS
    outs = pl.pallas_call(
        _group_body,
        grid_spec=pltpu.PrefetchScalarGridSpec(
            num_scalar_prefetch=5,
            grid=(NT,),
            in_specs=[
                pl.BlockSpec((_BR, INPUT_SIZE),
                             lambda i, tb, te, tf, rl, rh: (tb[i], 0)),
                pl.BlockSpec((1, HIDDEN_SIZE, INPUT_SIZE),
                             lambda i, tb, te, tf, rl, rh: (te[i], 0, 0)),
                pl.BlockSpec((1, HIDDEN_SIZE, INPUT_SIZE),
                             lambda i, tb, te, tf, rl, rh: (te[i], 0, 0)),
                pl.BlockSpec((1, OUTPUT_SIZE, HIDDEN_SIZE),
                             lambda i, tb, te, tf, rl, rh: (te[i], 0, 0)),
                pl.BlockSpec((1, 1, HIDDEN_SIZE),
                             lambda i, tb, te, tf, rl, rh: (te[i], 0, 0)),
                pl.BlockSpec((1, 1, HIDDEN_SIZE),
                             lambda i, tb, te, tf, rl, rh: (te[i], 0, 0)),
                pl.BlockSpec((1, 1, OUTPUT_SIZE),
                             lambda i, tb, te, tf, rl, rh: (te[i], 0, 0)),
            ],
            out_specs=pl.BlockSpec((_BR, OUTPUT_SIZE),
                                   lambda i, tb, te, tf, rl, rh: (tb[i], 0)),
        ),
        out_shape=jax.ShapeDtypeStruct((A, OUTPUT_SIZE), jnp.float32),
    )(tile_block, tile_expert, tile_first, row_lo, row_hi,
      xs, W_gate, W_up, W_down,
      b_gate[:, None, :], b_up[:, None, :], b_down[:, None, :])

    ncch = T // (_NW * _CCH)
    pA3c = pos[:, 0].reshape(_NW, ncch, _CCH)
    pB3c = pos[:, 1].reshape(_NW, ncch, _CCH)
    sA3 = jnp.broadcast_to(sc[:, 0:1], (T, 16)).reshape(_NW, ncch, _CCH, 16)
    sB3 = jnp.broadcast_to(sc[:, 1:2], (T, 16)).reshape(_NW, ncch, _CCH, 16)
    y = pl.kernel(
        _combine_body,
        out_type=jax.ShapeDtypeStruct((T, OUTPUT_SIZE), jnp.float32),
        mesh=mesh,
        compiler_params=pltpu.CompilerParams(use_tc_tiling_on_sc=False),
        scratch_types=[
            pltpu.VMEM((ncch, _CCH), jnp.int32),
            pltpu.VMEM((ncch, _CCH), jnp.int32),
            pltpu.VMEM((ncch, _CCH, 16), jnp.float32),
            pltpu.VMEM((ncch, _CCH, 16), jnp.float32),
            pltpu.VMEM((_CCH, OUTPUT_SIZE), jnp.float32),
            pltpu.VMEM((_CCH, OUTPUT_SIZE), jnp.float32),
            pltpu.SemaphoreType.DMA,
            pltpu.SemaphoreType.DMA,
        ],
    )(outs, pA3c, pB3c, sA3, sB3)

    return y.reshape(B, S, OUTPUT_SIZE), loss.reshape(())


# speedup vs baseline: 1.2924x; 1.2924x over previous
import functools

import jax
import jax.numpy as jnp
from jax import lax
from jax.experimental import pallas as pl
from jax.experimental.pallas import tpu as pltpu
from jax.experimental.pallas import tpu_sc as plsc

INPUT_SIZE = 1024
HIDDEN_SIZE = 176
OUTPUT_SIZE = 1024
NUM_EXPERTS = 64
NUM_SELECTS = 2
BALANCE_LOSS_WEIGHT = 1e-2

_GATE_BLK = 512
_BR = 256

_NC = 2
_NS = 16
_NW = _NC * _NS
_DCH = 32
_CCH = 16


def _gate_body(x_ref, g1_ref, g2_ref, idx_ref, sc_ref, rank_ref, cnt_ref,
               loss_ref, imp_ref, load_ref, carry_ref):
    tb = pl.program_id(0)
    nb = pl.num_programs(0)
    x = x_ref[...]
    t1 = jnp.tanh(jax.lax.dot_general(x, g1_ref[...], (((1,), (0,)), ((), ()))))
    logits = jax.lax.dot_general(t1, g2_ref[...], (((1,), (0,)), ((), ())))
    lane = jax.lax.broadcasted_iota(jnp.int32, logits.shape, 1)
    m1 = jnp.max(logits, axis=1, keepdims=True)
    i1 = jnp.min(jnp.where(logits == m1, lane, NUM_EXPERTS), axis=1, keepdims=True)
    mask1 = lane == i1
    l2 = jnp.where(mask1, -jnp.inf, logits)
    m2 = jnp.max(l2, axis=1, keepdims=True)
    i2 = jnp.min(jnp.where(l2 == m2, lane, NUM_EXPERTS), axis=1, keepdims=True)
    mask2 = lane == i2
    s1 = 1.0 / (1.0 + jnp.exp(m2 - m1))
    h1 = mask1.astype(jnp.float32)
    h2 = mask2.astype(jnp.float32)
    idx_ref[...] = jnp.concatenate([i1, i2], axis=1)
    sc_ref[...] = jnp.concatenate([s1, 1.0 - s1], axis=1)

    @pl.when(tb == 0)
    def _init():
        imp_ref[...] = jnp.zeros_like(imp_ref)
        load_ref[...] = jnp.zeros_like(load_ref)
        carry_ref[...] = jnp.zeros_like(carry_ref)

    r_io = jax.lax.broadcasted_iota(jnp.int32, (_GATE_BLK, _GATE_BLK), 0)
    c_io = jax.lax.broadcasted_iota(jnp.int32, (_GATE_BLK, _GATE_BLK), 1)
    ltri = (c_io < r_io).astype(jnp.bfloat16)
    hsum = h1 + h2
    cum = jax.lax.dot_general(ltri, hsum.astype(jnp.bfloat16),
                              (((1,), (0,)), ((), ())),
                              preferred_element_type=jnp.float32)
    carry = carry_ref[...]
    rank1 = jnp.sum(jnp.where(mask1, carry + cum, 0.0), axis=1, keepdims=True)
    rank2 = jnp.sum(jnp.where(mask2, carry + cum + h1, 0.0), axis=1,
                    keepdims=True)
    rank_ref[...] = jnp.concatenate([rank1, rank2], axis=1).astype(jnp.int32)
    carry_ref[...] = carry + jnp.sum(hsum, axis=0, keepdims=True)

    sf1 = jnp.where(mask1, s1, 0.0)
    sf2 = jnp.where(mask2, 1.0 - s1, 0.0)
    imp_ref[...] += jnp.sum(sf1 + sf2, axis=0, keepdims=True)
    load_ref[...] += (jnp.sum((sf1 > 0.0).astype(jnp.float32), axis=0, keepdims=True)
                      + jnp.sum((sf2 > 0.0).astype(jnp.float32), axis=0, keepdims=True))

    @pl.when(tb == nb - 1)
    def _fin():
        def cv2(v):
            mean = jnp.mean(v)
            var = jnp.sum((v - mean) ** 2) / (NUM_EXPERTS - 1)
            return var / (mean * mean + 1e-10)

        loss = BALANCE_LOSS_WEIGHT * (cv2(imp_ref[...]) + cv2(load_ref[...]))
        loss_ref[...] = loss * jnp.ones((1, 1), jnp.float32)
        cnt_ref[...] = carry_ref[...].astype(jnp.int32)


def _group_body(tb, te, tf, rl, rh, xs_ref, wg_ref, wu_ref, wd_ref,
                bg_ref, bu_ref, bd_ref, out_ref):
    i = pl.program_id(0)
    x = xs_ref[...]
    g = jax.lax.dot_general(x, wg_ref[0], (((1,), (1,)), ((), ()))) + bg_ref[0]
    u = jax.lax.dot_general(x, wu_ref[0], (((1,), (1,)), ((), ()))) + bu_ref[0]
    h = (g / (1.0 + jnp.exp(-g))) * u
    out = jax.lax.dot_general(h, wd_ref[0], (((1,), (1,)), ((), ()))) + bd_ref[0]
    rows = tb[i] * _BR + jax.lax.broadcasted_iota(jnp.int32, (_BR, 1), 0)
    m = ((rows >= rl[i]) & (rows < rh[i])).astype(jnp.float32)
    contrib = m * out

    @pl.when(tf[i] == 1)
    def _set():
        out_ref[...] = contrib

    @pl.when(tf[i] == 0)
    def _acc():
        out_ref[...] += contrib


def _routing_metadata(idx, sc, rank, cnt):
    A = idx.size
    counts = cnt.reshape(NUM_EXPERTS)
    bounds = jnp.concatenate(
        [jnp.zeros(1, jnp.int32), jnp.cumsum(counts, dtype=jnp.int32)])
    starts, ends = bounds[:NUM_EXPERTS], bounds[1:]
    sel = idx[..., None] == jnp.arange(NUM_EXPERTS, dtype=jnp.int32)
    pos = jnp.sum(jnp.where(sel, starts[None, None, :], 0),
                  axis=2).astype(jnp.int32) + rank
    NB = A // _BR
    edges0 = jnp.arange(NB, dtype=jnp.int32) * _BR
    edges1 = edges0 + (_BR - 1)
    eb_first = jnp.sum((ends[None, :] <= edges0[:, None]).astype(jnp.int32), axis=1)
    eb_last = jnp.sum((ends[None, :] <= edges1[:, None]).astype(jnp.int32), axis=1)
    nb = eb_last - eb_first + 1
    cum = jnp.concatenate([jnp.zeros(1, jnp.int32),
                           jnp.cumsum(nb, dtype=jnp.int32)])
    total = cum[NB]
    NT = NB + NUM_EXPERTS
    ti = jnp.arange(NT, dtype=jnp.int32)
    blk = jnp.clip(jnp.searchsorted(cum, ti, side="right") - 1, 0, NB - 1).astype(jnp.int32)
    pad = ti >= total
    tile_block = jnp.where(pad, NB - 1, blk)
    tile_expert = jnp.where(pad, eb_last[NB - 1],
                            eb_first[blk] + (ti - cum[blk]))
    row_lo = jnp.where(pad, 0, jnp.maximum(jnp.take(starts, tile_expert),
                                           tile_block * _BR))
    row_hi = jnp.where(pad, 0, jnp.minimum(jnp.take(ends, tile_expert),
                                           (tile_block + 1) * _BR))
    tile_first = ((ti == cum[blk]) & ~pad).astype(jnp.int32)
    return pos, tile_block, tile_expert, tile_first, row_lo, row_hi


def _dispatch_body(xf_hbm, pA_hbm, pB_hbm, xs_hbm, idxA_v, idxB_v, rows_v,
                   semA, semB):
    wid = lax.axis_index("s") * _NC + lax.axis_index("c")
    base = wid * (_DCH * (pA_hbm.shape[1]))
    pltpu.sync_copy(pA_hbm.at[wid], idxA_v)
    pltpu.sync_copy(pB_hbm.at[wid], idxB_v)
    for c in range(pA_hbm.shape[1]):
        pltpu.sync_copy(xf_hbm.at[pl.ds(base + c * _DCH, _DCH)], rows_v)
        a = pltpu.make_async_copy(rows_v, xs_hbm.at[idxA_v.at[c]], semA)
        b = pltpu.make_async_copy(rows_v, xs_hbm.at[idxB_v.at[c]], semB)
        a.start()
        b.start()
        a.wait()
        b.wait()


def _combine_body(outs_hbm, pA_hbm, pB_hbm, sA_hbm, sB_hbm, y_hbm,
                  idxA_v, idxB_v, sA_v, sB_v, a_v, b_v, semA, semB):
    wid = lax.axis_index("s") * _NC + lax.axis_index("c")
    nch = pA_hbm.shape[1]
    base = wid * (_CCH * nch)
    pltpu.sync_copy(pA_hbm.at[wid], idxA_v)
    pltpu.sync_copy(pB_hbm.at[wid], idxB_v)
    pltpu.sync_copy(sA_hbm.at[wid], sA_v)
    pltpu.sync_copy(sB_hbm.at[wid], sB_v)

    def chunk(c, carry):
        ga = pltpu.make_async_copy(outs_hbm.at[idxA_v.at[c]], a_v, semA)
        gb = pltpu.make_async_copy(outs_hbm.at[idxB_v.at[c]], b_v, semB)
        ga.start()
        gb.start()
        ga.wait()
        gb.wait()
        def col(k, cc):
            sl = pl.ds(k * 16, 16)
            for r in range(_CCH):
                a_v[r, sl] = (a_v[r, sl] * sA_v[c, pl.ds(r * 16, 16)]
                              + b_v[r, sl] * sB_v[c, pl.ds(r * 16, 16)])
            return cc

        lax.fori_loop(0, OUTPUT_SIZE // 16, col, 0)
        pltpu.sync_copy(a_v, y_hbm.at[pl.ds(base + c * _CCH, _CCH)])
        return carry

    lax.fori_loop(0, nch, chunk, 0)


def kernel(x, gate_w1, gate_w2, W_gate, W_up, W_down, b_gate, b_up, b_down):
    B, S, _ = x.shape
    T = B * S
    A = T * NUM_SELECTS
    xf = x.reshape(T, INPUT_SIZE)

    nbg = T // _GATE_BLK
    idx, sc, rank, cnt, loss = pl.pallas_call(
        _gate_body,
        grid=(nbg,),
        in_specs=[
            pl.BlockSpec((_GATE_BLK, INPUT_SIZE), lambda i: (i, 0)),
            pl.BlockSpec((INPUT_SIZE, NUM_EXPERTS), lambda i: (0, 0)),
            pl.BlockSpec((NUM_EXPERTS, NUM_EXPERTS), lambda i: (0, 0)),
        ],
        out_specs=[
            pl.BlockSpec((_GATE_BLK, NUM_SELECTS), lambda i: (i, 0)),
            pl.BlockSpec((_GATE_BLK, NUM_SELECTS), lambda i: (i, 0)),
            pl.BlockSpec((_GATE_BLK, NUM_SELECTS), lambda i: (i, 0)),
            pl.BlockSpec((1, NUM_EXPERTS), lambda i: (0, 0)),
            pl.BlockSpec((1, 1), lambda i: (0, 0)),
        ],
        out_shape=[
            jax.ShapeDtypeStruct((T, NUM_SELECTS), jnp.int32),
            jax.ShapeDtypeStruct((T, NUM_SELECTS), jnp.float32),
            jax.ShapeDtypeStruct((T, NUM_SELECTS), jnp.int32),
            jax.ShapeDtypeStruct((1, NUM_EXPERTS), jnp.int32),
            jax.ShapeDtypeStruct((1, 1), jnp.float32),
        ],
        scratch_shapes=[
            pltpu.VMEM((1, NUM_EXPERTS), jnp.float32),
            pltpu.VMEM((1, NUM_EXPERTS), jnp.float32),
            pltpu.VMEM((1, NUM_EXPERTS), jnp.float32),
        ],
    )(xf, gate_w1, gate_w2)

    (pos, tile_block, tile_expert, tile_first, row_lo,
     row_hi) = _routing_metadata(idx, sc, rank, cnt)

    mesh = plsc.VectorSubcoreMesh(core_axis_name="c", subcore_axis_name="s")
    ndch = T // (_NW * _DCH)
    pA3 = pos[:, 0].reshape(_NW, ndch, _DCH)
    pB3 = pos[:, 1].reshape(_NW, ndch, _DCH)
    xs = pl.kernel(
        _dispatch_body,
        out_type=jax.ShapeDtypeStruct((A, INPUT_SIZE), jnp.float32),
        mesh=mesh,
        compiler_params=pltpu.CompilerParams(use_tc_tiling_on_sc=True),
        scratch_types=[
            pltpu.VMEM((ndch, _DCH), jnp.int32),
            pltpu.VMEM((ndch, _DCH), jnp.int32),
            pltpu.VMEM((_DCH, INPUT_SIZE), jnp.float32),
            pltpu.SemaphoreType.DMA,
            pltpu.SemaphoreType.DMA,
        ],
    )(xf, pA3, pB3)

    NT = A // _BR + NUM_EXPERTS
    outs = pl.pallas_call(
        _group_body,
        grid_spec=pltpu.PrefetchScalarGridSpec(
            num_scalar_prefetch=5,
            grid=(NT,),
            in_specs=[
                pl.BlockSpec((_BR, INPUT_SIZE),
                             lambda i, tb, te, tf, rl, rh: (tb[i], 0)),
                pl.BlockSpec((1, HIDDEN_SIZE, INPUT_SIZE),
                             lambda i, tb, te, tf, rl, rh: (te[i], 0, 0)),
                pl.BlockSpec((1, HIDDEN_SIZE, INPUT_SIZE),
                             lambda i, tb, te, tf, rl, rh: (te[i], 0, 0)),
                pl.BlockSpec((1, OUTPUT_SIZE, HIDDEN_SIZE),
                             lambda i, tb, te, tf, rl, rh: (te[i], 0, 0)),
                pl.BlockSpec((1, 1, HIDDEN_SIZE),
                             lambda i, tb, te, tf, rl, rh: (te[i], 0, 0)),
                pl.BlockSpec((1, 1, HIDDEN_SIZE),
                             lambda i, tb, te, tf, rl, rh: (te[i], 0, 0)),
                pl.BlockSpec((1, 1, OUTPUT_SIZE),
                             lambda i, tb, te, tf, rl, rh: (te[i], 0, 0)),
            ],
            out_specs=pl.BlockSpec((_BR, OUTPUT_SIZE),
                                   lambda i, tb, te, tf, rl, rh: (tb[i], 0)),
        ),
        out_shape=jax.ShapeDtypeStruct((A, OUTPUT_SIZE), jnp.float32),
    )(tile_block, tile_expert, tile_first, row_lo, row_hi,
      xs, W_gate, W_up, W_down,
      b_gate[:, None, :], b_up[:, None, :], b_down[:, None, :])

    ncch = T // (_NW * _CCH)
    pA3c = pos[:, 0].reshape(_NW, ncch, _CCH)
    pB3c = pos[:, 1].reshape(_NW, ncch, _CCH)
    sA3 = jnp.broadcast_to(sc[:, 0:1], (T, 16)).reshape(_NW, ncch, _CCH * 16)
    sB3 = jnp.broadcast_to(sc[:, 1:2], (T, 16)).reshape(_NW, ncch, _CCH * 16)
    y = pl.kernel(
        _combine_body,
        out_type=jax.ShapeDtypeStruct((T, OUTPUT_SIZE), jnp.float32),
        mesh=mesh,
        compiler_params=pltpu.CompilerParams(use_tc_tiling_on_sc=True),
        scratch_types=[
            pltpu.VMEM((ncch, _CCH), jnp.int32),
            pltpu.VMEM((ncch, _CCH), jnp.int32),
            pltpu.VMEM((ncch, _CCH * 16), jnp.float32),
            pltpu.VMEM((ncch, _CCH * 16), jnp.float32),
            pltpu.VMEM((_CCH, OUTPUT_SIZE), jnp.float32),
            pltpu.VMEM((_CCH, OUTPUT_SIZE), jnp.float32),
            pltpu.SemaphoreType.DMA,
            pltpu.SemaphoreType.DMA,
        ],
    )(outs, pA3c, pB3c, sA3, sB3)

    return y.reshape(B, S, OUTPUT_SIZE), loss.reshape(())


# speedup vs baseline: 1.3582x; 1.0510x over previous
import functools

import jax
import jax.numpy as jnp
from jax import lax
from jax.experimental import pallas as pl
from jax.experimental.pallas import tpu as pltpu
from jax.experimental.pallas import tpu_sc as plsc

INPUT_SIZE = 1024
HIDDEN_SIZE = 176
OUTPUT_SIZE = 1024
NUM_EXPERTS = 64
NUM_SELECTS = 2
BALANCE_LOSS_WEIGHT = 1e-2

_GATE_BLK = 512
_BR = 256

_NC = 2
_NS = 16
_NW = _NC * _NS
_DCH = 32
_CCH = 16


def _gate_body(x_ref, g1_ref, g2_ref, idx_ref, sc_ref, rank_ref, cnt_ref,
               loss_ref, imp_ref, load_ref, carry_ref):
    tb = pl.program_id(0)
    nb = pl.num_programs(0)
    x = x_ref[...]
    t1 = jnp.tanh(jax.lax.dot_general(x, g1_ref[...], (((1,), (0,)), ((), ()))))
    logits = jax.lax.dot_general(t1, g2_ref[...], (((1,), (0,)), ((), ())))
    lane = jax.lax.broadcasted_iota(jnp.int32, logits.shape, 1)
    m1 = jnp.max(logits, axis=1, keepdims=True)
    i1 = jnp.min(jnp.where(logits == m1, lane, NUM_EXPERTS), axis=1, keepdims=True)
    mask1 = lane == i1
    l2 = jnp.where(mask1, -jnp.inf, logits)
    m2 = jnp.max(l2, axis=1, keepdims=True)
    i2 = jnp.min(jnp.where(l2 == m2, lane, NUM_EXPERTS), axis=1, keepdims=True)
    mask2 = lane == i2
    s1 = 1.0 / (1.0 + jnp.exp(m2 - m1))
    h1 = mask1.astype(jnp.float32)
    h2 = mask2.astype(jnp.float32)
    idx_ref[...] = jnp.concatenate([i1, i2], axis=1)
    sc_ref[...] = jnp.concatenate([s1, 1.0 - s1], axis=1)

    @pl.when(tb == 0)
    def _init():
        imp_ref[...] = jnp.zeros_like(imp_ref)
        load_ref[...] = jnp.zeros_like(load_ref)
        carry_ref[...] = jnp.zeros_like(carry_ref)

    r_io = jax.lax.broadcasted_iota(jnp.int32, (_GATE_BLK, _GATE_BLK), 0)
    c_io = jax.lax.broadcasted_iota(jnp.int32, (_GATE_BLK, _GATE_BLK), 1)
    ltri = (c_io < r_io).astype(jnp.bfloat16)
    hsum = h1 + h2
    cum = jax.lax.dot_general(ltri, hsum.astype(jnp.bfloat16),
                              (((1,), (0,)), ((), ())),
                              preferred_element_type=jnp.float32)
    carry = carry_ref[...]
    rank1 = jnp.sum(jnp.where(mask1, carry + cum, 0.0), axis=1, keepdims=True)
    rank2 = jnp.sum(jnp.where(mask2, carry + cum + h1, 0.0), axis=1,
                    keepdims=True)
    rank_ref[...] = jnp.concatenate([rank1, rank2], axis=1).astype(jnp.int32)
    carry_ref[...] = carry + jnp.sum(hsum, axis=0, keepdims=True)

    sf1 = jnp.where(mask1, s1, 0.0)
    sf2 = jnp.where(mask2, 1.0 - s1, 0.0)
    imp_ref[...] += jnp.sum(sf1 + sf2, axis=0, keepdims=True)
    load_ref[...] += (jnp.sum((sf1 > 0.0).astype(jnp.float32), axis=0, keepdims=True)
                      + jnp.sum((sf2 > 0.0).astype(jnp.float32), axis=0, keepdims=True))

    @pl.when(tb == nb - 1)
    def _fin():
        def cv2(v):
            mean = jnp.mean(v)
            var = jnp.sum((v - mean) ** 2) / (NUM_EXPERTS - 1)
            return var / (mean * mean + 1e-10)

        loss = BALANCE_LOSS_WEIGHT * (cv2(imp_ref[...]) + cv2(load_ref[...]))
        loss_ref[...] = loss * jnp.ones((1, 1), jnp.float32)
        cnt_ref[...] = carry_ref[...].astype(jnp.int32)


def _group_body(tb, te, tf, rl, rh, xs_ref, wg_ref, wu_ref, wd_ref,
                bg_ref, bu_ref, bd_ref, out_ref):
    i = pl.program_id(0)
    x = xs_ref[...]
    g = jax.lax.dot_general(x, wg_ref[0], (((1,), (1,)), ((), ()))) + bg_ref[0]
    u = jax.lax.dot_general(x, wu_ref[0], (((1,), (1,)), ((), ()))) + bu_ref[0]
    h = (g / (1.0 + jnp.exp(-g))) * u
    out = jax.lax.dot_general(h, wd_ref[0], (((1,), (1,)), ((), ()))) + bd_ref[0]
    rows = tb[i] * _BR + jax.lax.broadcasted_iota(jnp.int32, (_BR, 1), 0)
    m = ((rows >= rl[i]) & (rows < rh[i])).astype(jnp.float32)
    contrib = m * out

    @pl.when(tf[i] == 1)
    def _set():
        out_ref[...] = contrib

    @pl.when(tf[i] == 0)
    def _acc():
        out_ref[...] += contrib


def _routing_metadata(idx, sc, rank, cnt):
    A = idx.size
    counts = cnt.reshape(NUM_EXPERTS)
    bounds = jnp.concatenate(
        [jnp.zeros(1, jnp.int32), jnp.cumsum(counts, dtype=jnp.int32)])
    starts, ends = bounds[:NUM_EXPERTS], bounds[1:]
    sel = idx[..., None] == jnp.arange(NUM_EXPERTS, dtype=jnp.int32)
    pos = jnp.sum(jnp.where(sel, starts[None, None, :], 0),
                  axis=2).astype(jnp.int32) + rank
    NB = A // _BR
    edges0 = jnp.arange(NB, dtype=jnp.int32) * _BR
    edges1 = edges0 + (_BR - 1)
    eb_first = jnp.sum((ends[None, :] <= edges0[:, None]).astype(jnp.int32), axis=1)
    eb_last = jnp.sum((ends[None, :] <= edges1[:, None]).astype(jnp.int32), axis=1)
    nb = eb_last - eb_first + 1
    cum = jnp.concatenate([jnp.zeros(1, jnp.int32),
                           jnp.cumsum(nb, dtype=jnp.int32)])
    total = cum[NB]
    NT = NB + NUM_EXPERTS
    ti = jnp.arange(NT, dtype=jnp.int32)
    blk = jnp.clip(jnp.searchsorted(cum, ti, side="right") - 1, 0, NB - 1).astype(jnp.int32)
    pad = ti >= total
    tile_block = jnp.where(pad, NB - 1, blk)
    tile_expert = jnp.where(pad, eb_last[NB - 1],
                            eb_first[blk] + (ti - cum[blk]))
    row_lo = jnp.where(pad, 0, jnp.maximum(jnp.take(starts, tile_expert),
                                           tile_block * _BR))
    row_hi = jnp.where(pad, 0, jnp.minimum(jnp.take(ends, tile_expert),
                                           (tile_block + 1) * _BR))
    tile_first = ((ti == cum[blk]) & ~pad).astype(jnp.int32)
    return pos, tile_block, tile_expert, tile_first, row_lo, row_hi


def _dispatch_body(xf_hbm, pA_hbm, pB_hbm, xs_hbm, idxA_v, idxB_v, rows_v,
                   semA, semB):
    wid = lax.axis_index("s") * _NC + lax.axis_index("c")
    base = wid * (_DCH * (pA_hbm.shape[1]))
    pltpu.sync_copy(pA_hbm.at[wid], idxA_v)
    pltpu.sync_copy(pB_hbm.at[wid], idxB_v)
    for c in range(pA_hbm.shape[1]):
        pltpu.sync_copy(xf_hbm.at[pl.ds(base + c * _DCH, _DCH)], rows_v)
        a = pltpu.make_async_copy(rows_v, xs_hbm.at[idxA_v.at[c]], semA)
        b = pltpu.make_async_copy(rows_v, xs_hbm.at[idxB_v.at[c]], semB)
        a.start()
        b.start()
        a.wait()
        b.wait()


def _combine_body(outs_hbm, pA_hbm, pB_hbm, sA_hbm, sB_hbm, y_hbm,
                  idxA_v, idxB_v, sA_v, sB_v, a0_v, b0_v, a1_v, b1_v,
                  semA0, semB0, semA1, semB1):
    wid = lax.axis_index("s") * _NC + lax.axis_index("c")
    nch = pA_hbm.shape[1]
    base = wid * (_CCH * nch)
    pltpu.sync_copy(pA_hbm.at[wid], idxA_v)
    pltpu.sync_copy(pB_hbm.at[wid], idxB_v)
    pltpu.sync_copy(sA_hbm.at[wid], sA_v)
    pltpu.sync_copy(sB_hbm.at[wid], sB_v)
    bufs = [(a0_v, b0_v, semA0, semB0), (a1_v, b1_v, semA1, semB1)]

    def gathers(c):
        av, bv, sA, sB = bufs[c % 2]
        return (pltpu.make_async_copy(outs_hbm.at[idxA_v.at[c]], av, sA),
                pltpu.make_async_copy(outs_hbm.at[idxB_v.at[c]], bv, sB))

    for d in gathers(0):
        d.start()
    for c in range(nch):
        if c + 1 < nch:
            for d in gathers(c + 1):
                d.start()
        av, bv, _, _ = bufs[c % 2]
        for d in gathers(c):
            d.wait()

        def col(k, cc):
            sl = pl.ds(k * 16, 16)
            for r in range(_CCH):
                av[r, sl] = (av[r, sl] * sA_v[c, pl.ds(r * 16, 16)]
                             + bv[r, sl] * sB_v[c, pl.ds(r * 16, 16)])
            return cc

        lax.fori_loop(0, OUTPUT_SIZE // 16, col, 0)
        pltpu.sync_copy(av, y_hbm.at[pl.ds(base + c * _CCH, _CCH)])


def kernel(x, gate_w1, gate_w2, W_gate, W_up, W_down, b_gate, b_up, b_down):
    B, S, _ = x.shape
    T = B * S
    A = T * NUM_SELECTS
    xf = x.reshape(T, INPUT_SIZE)

    nbg = T // _GATE_BLK
    idx, sc, rank, cnt, loss = pl.pallas_call(
        _gate_body,
        grid=(nbg,),
        in_specs=[
            pl.BlockSpec((_GATE_BLK, INPUT_SIZE), lambda i: (i, 0)),
            pl.BlockSpec((INPUT_SIZE, NUM_EXPERTS), lambda i: (0, 0)),
            pl.BlockSpec((NUM_EXPERTS, NUM_EXPERTS), lambda i: (0, 0)),
        ],
        out_specs=[
            pl.BlockSpec((_GATE_BLK, NUM_SELECTS), lambda i: (i, 0)),
            pl.BlockSpec((_GATE_BLK, NUM_SELECTS), lambda i: (i, 0)),
            pl.BlockSpec((_GATE_BLK, NUM_SELECTS), lambda i: (i, 0)),
            pl.BlockSpec((1, NUM_EXPERTS), lambda i: (0, 0)),
            pl.BlockSpec((1, 1), lambda i: (0, 0)),
        ],
        out_shape=[
            jax.ShapeDtypeStruct((T, NUM_SELECTS), jnp.int32),
            jax.ShapeDtypeStruct((T, NUM_SELECTS), jnp.float32),
            jax.ShapeDtypeStruct((T, NUM_SELECTS), jnp.int32),
            jax.ShapeDtypeStruct((1, NUM_EXPERTS), jnp.int32),
            jax.ShapeDtypeStruct((1, 1), jnp.float32),
        ],
        scratch_shapes=[
            pltpu.VMEM((1, NUM_EXPERTS), jnp.float32),
            pltpu.VMEM((1, NUM_EXPERTS), jnp.float32),
            pltpu.VMEM((1, NUM_EXPERTS), jnp.float32),
        ],
    )(xf, gate_w1, gate_w2)

    (pos, tile_block, tile_expert, tile_first, row_lo,
     row_hi) = _routing_metadata(idx, sc, rank, cnt)

    mesh = plsc.VectorSubcoreMesh(core_axis_name="c", subcore_axis_name="s")
    ndch = T // (_NW * _DCH)
    pA3 = pos[:, 0].reshape(_NW, ndch, _DCH)
    pB3 = pos[:, 1].reshape(_NW, ndch, _DCH)
    xs = pl.kernel(
        _dispatch_body,
        out_type=jax.ShapeDtypeStruct((A, INPUT_SIZE), jnp.float32),
        mesh=mesh,
        compiler_params=pltpu.CompilerParams(use_tc_tiling_on_sc=True),
        scratch_types=[
            pltpu.VMEM((ndch, _DCH), jnp.int32),
            pltpu.VMEM((ndch, _DCH), jnp.int32),
            pltpu.VMEM((_DCH, INPUT_SIZE), jnp.float32),
            pltpu.SemaphoreType.DMA,
            pltpu.SemaphoreType.DMA,
        ],
    )(xf, pA3, pB3)

    NT = A // _BR + NUM_EXPERTS
    outs = pl.pallas_call(
        _group_body,
        grid_spec=pltpu.PrefetchScalarGridSpec(
            num_scalar_prefetch=5,
            grid=(NT,),
            in_specs=[
                pl.BlockSpec((_BR, INPUT_SIZE),
                             lambda i, tb, te, tf, rl, rh: (tb[i], 0)),
                pl.BlockSpec((1, HIDDEN_SIZE, INPUT_SIZE),
                             lambda i, tb, te, tf, rl, rh: (te[i], 0, 0)),
                pl.BlockSpec((1, HIDDEN_SIZE, INPUT_SIZE),
                             lambda i, tb, te, tf, rl, rh: (te[i], 0, 0)),
                pl.BlockSpec((1, OUTPUT_SIZE, HIDDEN_SIZE),
                             lambda i, tb, te, tf, rl, rh: (te[i], 0, 0)),
                pl.BlockSpec((1, 1, HIDDEN_SIZE),
                             lambda i, tb, te, tf, rl, rh: (te[i], 0, 0)),
                pl.BlockSpec((1, 1, HIDDEN_SIZE),
                             lambda i, tb, te, tf, rl, rh: (te[i], 0, 0)),
                pl.BlockSpec((1, 1, OUTPUT_SIZE),
                             lambda i, tb, te, tf, rl, rh: (te[i], 0, 0)),
            ],
            out_specs=pl.BlockSpec((_BR, OUTPUT_SIZE),
                                   lambda i, tb, te, tf, rl, rh: (tb[i], 0)),
        ),
        out_shape=jax.ShapeDtypeStruct((A, OUTPUT_SIZE), jnp.float32),
    )(tile_block, tile_expert, tile_first, row_lo, row_hi,
      xs, W_gate, W_up, W_down,
      b_gate[:, None, :], b_up[:, None, :], b_down[:, None, :])

    ncch = T // (_NW * _CCH)
    pA3c = pos[:, 0].reshape(_NW, ncch, _CCH)
    pB3c = pos[:, 1].reshape(_NW, ncch, _CCH)
    sA3 = jnp.broadcast_to(sc[:, 0:1], (T, 16)).reshape(_NW, ncch, _CCH * 16)
    sB3 = jnp.broadcast_to(sc[:, 1:2], (T, 16)).reshape(_NW, ncch, _CCH * 16)
    y = pl.kernel(
        _combine_body,
        out_type=jax.ShapeDtypeStruct((T, OUTPUT_SIZE), jnp.float32),
        mesh=mesh,
        compiler_params=pltpu.CompilerParams(use_tc_tiling_on_sc=True),
        scratch_types=[
            pltpu.VMEM((ncch, _CCH), jnp.int32),
            pltpu.VMEM((ncch, _CCH), jnp.int32),
            pltpu.VMEM((ncch, _CCH * 16), jnp.float32),
            pltpu.VMEM((ncch, _CCH * 16), jnp.float32),
            pltpu.VMEM((_CCH, OUTPUT_SIZE), jnp.float32),
            pltpu.VMEM((_CCH, OUTPUT_SIZE), jnp.float32),
            pltpu.VMEM((_CCH, OUTPUT_SIZE), jnp.float32),
            pltpu.VMEM((_CCH, OUTPUT_SIZE), jnp.float32),
            pltpu.SemaphoreType.DMA,
            pltpu.SemaphoreType.DMA,
            pltpu.SemaphoreType.DMA,
            pltpu.SemaphoreType.DMA,
        ],
    )(outs, pA3c, pB3c, sA3, sB3)

    return y.reshape(B, S, OUTPUT_SIZE), loss.reshape(())


# speedup vs baseline: 1.3587x; 1.0004x over previous
import functools

import jax
import jax.numpy as jnp
from jax import lax
from jax.experimental import pallas as pl
from jax.experimental.pallas import tpu as pltpu
from jax.experimental.pallas import tpu_sc as plsc

INPUT_SIZE = 1024
HIDDEN_SIZE = 176
OUTPUT_SIZE = 1024
NUM_EXPERTS = 64
NUM_SELECTS = 2
BALANCE_LOSS_WEIGHT = 1e-2

_GATE_BLK = 512
_BR = 256

_NC = 2
_NS = 16
_NW = _NC * _NS
_DCH = 32
_CCH = 16


def _gate_body(x_ref, g1_ref, g2_ref, idx_ref, sc_ref, rank_ref, cnt_ref,
               loss_ref, imp_ref, load_ref, carry_ref):
    tb = pl.program_id(0)
    nb = pl.num_programs(0)
    x = x_ref[...]
    t1 = jnp.tanh(jax.lax.dot_general(x, g1_ref[...], (((1,), (0,)), ((), ()))))
    logits = jax.lax.dot_general(t1, g2_ref[...], (((1,), (0,)), ((), ())))
    lane = jax.lax.broadcasted_iota(jnp.int32, logits.shape, 1)
    m1 = jnp.max(logits, axis=1, keepdims=True)
    i1 = jnp.min(jnp.where(logits == m1, lane, NUM_EXPERTS), axis=1, keepdims=True)
    mask1 = lane == i1
    l2 = jnp.where(mask1, -jnp.inf, logits)
    m2 = jnp.max(l2, axis=1, keepdims=True)
    i2 = jnp.min(jnp.where(l2 == m2, lane, NUM_EXPERTS), axis=1, keepdims=True)
    mask2 = lane == i2
    s1 = 1.0 / (1.0 + jnp.exp(m2 - m1))
    h1 = mask1.astype(jnp.float32)
    h2 = mask2.astype(jnp.float32)
    idx_ref[...] = jnp.concatenate([i1, i2], axis=1)
    sc_ref[...] = jnp.concatenate([s1, 1.0 - s1], axis=1)

    @pl.when(tb == 0)
    def _init():
        imp_ref[...] = jnp.zeros_like(imp_ref)
        load_ref[...] = jnp.zeros_like(load_ref)
        carry_ref[...] = jnp.zeros_like(carry_ref)

    r_io = jax.lax.broadcasted_iota(jnp.int32, (_GATE_BLK, _GATE_BLK), 0)
    c_io = jax.lax.broadcasted_iota(jnp.int32, (_GATE_BLK, _GATE_BLK), 1)
    ltri = (c_io < r_io).astype(jnp.bfloat16)
    hsum = h1 + h2
    cum = jax.lax.dot_general(ltri, hsum.astype(jnp.bfloat16),
                              (((1,), (0,)), ((), ())),
                              preferred_element_type=jnp.float32)
    carry = carry_ref[...]
    rank1 = jnp.sum(jnp.where(mask1, carry + cum, 0.0), axis=1, keepdims=True)
    rank2 = jnp.sum(jnp.where(mask2, carry + cum + h1, 0.0), axis=1,
                    keepdims=True)
    rank_ref[...] = jnp.concatenate([rank1, rank2], axis=1).astype(jnp.int32)
    carry_ref[...] = carry + jnp.sum(hsum, axis=0, keepdims=True)

    sf1 = jnp.where(mask1, s1, 0.0)
    sf2 = jnp.where(mask2, 1.0 - s1, 0.0)
    imp_ref[...] += jnp.sum(sf1 + sf2, axis=0, keepdims=True)
    load_ref[...] += (jnp.sum((sf1 > 0.0).astype(jnp.float32), axis=0, keepdims=True)
                      + jnp.sum((sf2 > 0.0).astype(jnp.float32), axis=0, keepdims=True))

    @pl.when(tb == nb - 1)
    def _fin():
        def cv2(v):
            mean = jnp.mean(v)
            var = jnp.sum((v - mean) ** 2) / (NUM_EXPERTS - 1)
            return var / (mean * mean + 1e-10)

        loss = BALANCE_LOSS_WEIGHT * (cv2(imp_ref[...]) + cv2(load_ref[...]))
        loss_ref[...] = loss * jnp.ones((1, 1), jnp.float32)
        cnt_ref[...] = carry_ref[...].astype(jnp.int32)


def _group_body(tb, te, tf, rl, rh, xs_ref, wg_ref, wu_ref, wd_ref,
                bg_ref, bu_ref, bd_ref, out_ref):
    i = pl.program_id(0)
    x = xs_ref[...]
    g = jax.lax.dot_general(x, wg_ref[0], (((1,), (1,)), ((), ()))) + bg_ref[0]
    u = jax.lax.dot_general(x, wu_ref[0], (((1,), (1,)), ((), ()))) + bu_ref[0]
    h = (g / (1.0 + jnp.exp(-g))) * u
    out = jax.lax.dot_general(h, wd_ref[0], (((1,), (1,)), ((), ()))) + bd_ref[0]
    rows = tb[i] * _BR + jax.lax.broadcasted_iota(jnp.int32, (_BR, 1), 0)
    m = ((rows >= rl[i]) & (rows < rh[i])).astype(jnp.float32)
    contrib = m * out

    @pl.when(tf[i] == 1)
    def _set():
        out_ref[...] = contrib

    @pl.when(tf[i] == 0)
    def _acc():
        out_ref[...] += contrib


def _routing_metadata(idx, sc, rank, cnt):
    A = idx.size
    counts = cnt.reshape(NUM_EXPERTS)
    bounds = jnp.concatenate(
        [jnp.zeros(1, jnp.int32), jnp.cumsum(counts, dtype=jnp.int32)])
    starts, ends = bounds[:NUM_EXPERTS], bounds[1:]
    sel = idx[..., None] == jnp.arange(NUM_EXPERTS, dtype=jnp.int32)
    pos = jnp.sum(jnp.where(sel, starts[None, None, :], 0),
                  axis=2).astype(jnp.int32) + rank
    NB = A // _BR
    edges0 = jnp.arange(NB, dtype=jnp.int32) * _BR
    edges1 = edges0 + (_BR - 1)
    eb_first = jnp.sum((ends[None, :] <= edges0[:, None]).astype(jnp.int32), axis=1)
    eb_last = jnp.sum((ends[None, :] <= edges1[:, None]).astype(jnp.int32), axis=1)
    nb = eb_last - eb_first + 1
    cum = jnp.concatenate([jnp.zeros(1, jnp.int32),
                           jnp.cumsum(nb, dtype=jnp.int32)])
    total = cum[NB]
    NT = NB + NUM_EXPERTS
    ti = jnp.arange(NT, dtype=jnp.int32)
    blk = jnp.clip(jnp.searchsorted(cum, ti, side="right") - 1, 0, NB - 1).astype(jnp.int32)
    pad = ti >= total
    tile_block = jnp.where(pad, NB - 1, blk)
    tile_expert = jnp.where(pad, eb_last[NB - 1],
                            eb_first[blk] + (ti - cum[blk]))
    row_lo = jnp.where(pad, 0, jnp.maximum(jnp.take(starts, tile_expert),
                                           tile_block * _BR))
    row_hi = jnp.where(pad, 0, jnp.minimum(jnp.take(ends, tile_expert),
                                           (tile_block + 1) * _BR))
    tile_first = ((ti == cum[blk]) & ~pad).astype(jnp.int32)
    return pos, tile_block, tile_expert, tile_first, row_lo, row_hi


def _dispatch_body(xf_hbm, pA_hbm, pB_hbm, xs_hbm, idxA_v, idxB_v,
                   r0_v, r1_v, semL0, semL1, semA0, semB0, semA1, semB1):
    wid = lax.axis_index("s") * _NC + lax.axis_index("c")
    nch = pA_hbm.shape[1]
    base = wid * (_DCH * nch)
    pltpu.sync_copy(pA_hbm.at[wid], idxA_v)
    pltpu.sync_copy(pB_hbm.at[wid], idxB_v)
    rows = [(r0_v, semL0, semA0, semB0), (r1_v, semL1, semA1, semB1)]

    def load(c):
        rv, sl, _, _ = rows[c % 2]
        return pltpu.make_async_copy(
            xf_hbm.at[pl.ds(base + c * _DCH, _DCH)], rv, sl)

    scat = [None, None]
    load(0).start()
    for c in range(nch):
        rv, _, sa, sb = rows[c % 2]
        load(c).wait()
        a = pltpu.make_async_copy(rv, xs_hbm.at[idxA_v.at[c]], sa)
        b = pltpu.make_async_copy(rv, xs_hbm.at[idxB_v.at[c]], sb)
        a.start()
        b.start()
        scat[c % 2] = (a, b)
        nxt = c + 1
        if nxt < nch:
            if scat[nxt % 2] is not None:
                for d in scat[nxt % 2]:
                    d.wait()
            load(nxt).start()
    for pair in scat:
        if pair is not None:
            for d in pair:
                d.wait()


def _combine_body(outs_hbm, pA_hbm, pB_hbm, sA_hbm, sB_hbm, y_hbm,
                  idxA_v, idxB_v, sA_v, sB_v, a0_v, b0_v, a1_v, b1_v,
                  semA0, semB0, semA1, semB1):
    wid = lax.axis_index("s") * _NC + lax.axis_index("c")
    nch = pA_hbm.shape[1]
    base = wid * (_CCH * nch)
    pltpu.sync_copy(pA_hbm.at[wid], idxA_v)
    pltpu.sync_copy(pB_hbm.at[wid], idxB_v)
    pltpu.sync_copy(sA_hbm.at[wid], sA_v)
    pltpu.sync_copy(sB_hbm.at[wid], sB_v)
    bufs = [(a0_v, b0_v, semA0, semB0), (a1_v, b1_v, semA1, semB1)]

    def gathers(c):
        av, bv, sA, sB = bufs[c % 2]
        return (pltpu.make_async_copy(outs_hbm.at[idxA_v.at[c]], av, sA),
                pltpu.make_async_copy(outs_hbm.at[idxB_v.at[c]], bv, sB))

    for d in gathers(0):
        d.start()
    for c in range(nch):
        if c + 1 < nch:
            for d in gathers(c + 1):
                d.start()
        av, bv, _, _ = bufs[c % 2]
        for d in gathers(c):
            d.wait()

        def col(k, cc):
            sl = pl.ds(k * 16, 16)
            for r in range(_CCH):
                av[r, sl] = (av[r, sl] * sA_v[c, pl.ds(r * 16, 16)]
                             + bv[r, sl] * sB_v[c, pl.ds(r * 16, 16)])
            return cc

        lax.fori_loop(0, OUTPUT_SIZE // 16, col, 0)
        pltpu.sync_copy(av, y_hbm.at[pl.ds(base + c * _CCH, _CCH)])


def kernel(x, gate_w1, gate_w2, W_gate, W_up, W_down, b_gate, b_up, b_down):
    B, S, _ = x.shape
    T = B * S
    A = T * NUM_SELECTS
    xf = x.reshape(T, INPUT_SIZE)

    nbg = T // _GATE_BLK
    idx, sc, rank, cnt, loss = pl.pallas_call(
        _gate_body,
        grid=(nbg,),
        in_specs=[
            pl.BlockSpec((_GATE_BLK, INPUT_SIZE), lambda i: (i, 0)),
            pl.BlockSpec((INPUT_SIZE, NUM_EXPERTS), lambda i: (0, 0)),
            pl.BlockSpec((NUM_EXPERTS, NUM_EXPERTS), lambda i: (0, 0)),
        ],
        out_specs=[
            pl.BlockSpec((_GATE_BLK, NUM_SELECTS), lambda i: (i, 0)),
            pl.BlockSpec((_GATE_BLK, NUM_SELECTS), lambda i: (i, 0)),
            pl.BlockSpec((_GATE_BLK, NUM_SELECTS), lambda i: (i, 0)),
            pl.BlockSpec((1, NUM_EXPERTS), lambda i: (0, 0)),
            pl.BlockSpec((1, 1), lambda i: (0, 0)),
        ],
        out_shape=[
            jax.ShapeDtypeStruct((T, NUM_SELECTS), jnp.int32),
            jax.ShapeDtypeStruct((T, NUM_SELECTS), jnp.float32),
            jax.ShapeDtypeStruct((T, NUM_SELECTS), jnp.int32),
            jax.ShapeDtypeStruct((1, NUM_EXPERTS), jnp.int32),
            jax.ShapeDtypeStruct((1, 1), jnp.float32),
        ],
        scratch_shapes=[
            pltpu.VMEM((1, NUM_EXPERTS), jnp.float32),
            pltpu.VMEM((1, NUM_EXPERTS), jnp.float32),
            pltpu.VMEM((1, NUM_EXPERTS), jnp.float32),
        ],
    )(xf, gate_w1, gate_w2)

    (pos, tile_block, tile_expert, tile_first, row_lo,
     row_hi) = _routing_metadata(idx, sc, rank, cnt)

    mesh = plsc.VectorSubcoreMesh(core_axis_name="c", subcore_axis_name="s")
    ndch = T // (_NW * _DCH)
    pA3 = pos[:, 0].reshape(_NW, ndch, _DCH)
    pB3 = pos[:, 1].reshape(_NW, ndch, _DCH)
    xs = pl.kernel(
        _dispatch_body,
        out_type=jax.ShapeDtypeStruct((A, INPUT_SIZE), jnp.float32),
        mesh=mesh,
        compiler_params=pltpu.CompilerParams(use_tc_tiling_on_sc=True),
        scratch_types=[
            pltpu.VMEM((ndch, _DCH), jnp.int32),
            pltpu.VMEM((ndch, _DCH), jnp.int32),
            pltpu.VMEM((_DCH, INPUT_SIZE), jnp.float32),
            pltpu.VMEM((_DCH, INPUT_SIZE), jnp.float32),
            pltpu.SemaphoreType.DMA,
            pltpu.SemaphoreType.DMA,
            pltpu.SemaphoreType.DMA,
            pltpu.SemaphoreType.DMA,
            pltpu.SemaphoreType.DMA,
            pltpu.SemaphoreType.DMA,
        ],
    )(xf, pA3, pB3)

    NT = A // _BR + NUM_EXPERTS
    outs = pl.pallas_call(
        _group_body,
        grid_spec=pltpu.PrefetchScalarGridSpec(
            num_scalar_prefetch=5,
            grid=(NT,),
            in_specs=[
                pl.BlockSpec((_BR, INPUT_SIZE),
                             lambda i, tb, te, tf, rl, rh: (tb[i], 0)),
                pl.BlockSpec((1, HIDDEN_SIZE, INPUT_SIZE),
                             lambda i, tb, te, tf, rl, rh: (te[i], 0, 0)),
                pl.BlockSpec((1, HIDDEN_SIZE, INPUT_SIZE),
                             lambda i, tb, te, tf, rl, rh: (te[i], 0, 0)),
                pl.BlockSpec((1, OUTPUT_SIZE, HIDDEN_SIZE),
                             lambda i, tb, te, tf, rl, rh: (te[i], 0, 0)),
                pl.BlockSpec((1, 1, HIDDEN_SIZE),
                             lambda i, tb, te, tf, rl, rh: (te[i], 0, 0)),
                pl.BlockSpec((1, 1, HIDDEN_SIZE),
                             lambda i, tb, te, tf, rl, rh: (te[i], 0, 0)),
                pl.BlockSpec((1, 1, OUTPUT_SIZE),
                             lambda i, tb, te, tf, rl, rh: (te[i], 0, 0)),
            ],
            out_specs=pl.BlockSpec((_BR, OUTPUT_SIZE),
                                   lambda i, tb, te, tf, rl, rh: (tb[i], 0)),
        ),
        out_shape=jax.ShapeDtypeStruct((A, OUTPUT_SIZE), jnp.float32),
    )(tile_block, tile_expert, tile_first, row_lo, row_hi,
      xs, W_gate, W_up, W_down,
      b_gate[:, None, :], b_up[:, None, :], b_down[:, None, :])

    ncch = T // (_NW * _CCH)
    pA3c = pos[:, 0].reshape(_NW, ncch, _CCH)
    pB3c = pos[:, 1].reshape(_NW, ncch, _CCH)
    sA3 = jnp.broadcast_to(sc[:, 0:1], (T, 16)).reshape(_NW, ncch, _CCH * 16)
    sB3 = jnp.broadcast_to(sc[:, 1:2], (T, 16)).reshape(_NW, ncch, _CCH * 16)
    y = pl.kernel(
        _combine_body,
        out_type=jax.ShapeDtypeStruct((T, OUTPUT_SIZE), jnp.float32),
        mesh=mesh,
        compiler_params=pltpu.CompilerParams(use_tc_tiling_on_sc=True),
        scratch_types=[
            pltpu.VMEM((ncch, _CCH), jnp.int32),
            pltpu.VMEM((ncch, _CCH), jnp.int32),
            pltpu.VMEM((ncch, _CCH * 16), jnp.float32),
            pltpu.VMEM((ncch, _CCH * 16), jnp.float32),
            pltpu.VMEM((_CCH, OUTPUT_SIZE), jnp.float32),
            pltpu.VMEM((_CCH, OUTPUT_SIZE), jnp.float32),
            pltpu.VMEM((_CCH, OUTPUT_SIZE), jnp.float32),
            pltpu.VMEM((_CCH, OUTPUT_SIZE), jnp.float32),
            pltpu.SemaphoreType.DMA,
            pltpu.SemaphoreType.DMA,
            pltpu.SemaphoreType.DMA,
            pltpu.SemaphoreType.DMA,
        ],
    )(outs, pA3c, pB3c, sA3, sB3)

    return y.reshape(B, S, OUTPUT_SIZE), loss.reshape(())
